# baseline (speedup 1.0000x reference)
import functools

import jax
import jax.numpy as jnp
from jax import lax
from jax.experimental import pallas as pl
from jax.experimental.pallas import tpu as pltpu
from jax.experimental.pallas import tpu_sc as plsc

_HID = 64
_BATCH = 4096
_SEQ = 24576
_MARGIN = 1.0

_NC = 2
_NS = 16
_L = 16
_NW = _NC * _NS
_ROWS_W = _SEQ // _NW
_CHUNK = 128
_NCH = _ROWS_W // _CHUNK
_NSEG = _HID // _L
_W = 2 * _HID

_ENT = 1000000
_TBLK = 32768
_TH = _TBLK // 2
_SHIFT_HI = 15
_SHIFT_LO = 14
_TGRID = pl.cdiv(_ENT, _TBLK)
_PROWS = _TGRID * _TH


def _permute(v, idx):
  dnums = lax.GatherDimensionNumbers(
      offset_dims=(), collapsed_slice_dims=(0,), start_index_map=(0,))
  return lax.gather(v, idx[:, None], dnums, (1,),
                    mode=lax.GatherScatterMode.PROMISE_IN_BOUNDS)


def _transpose_body(in_ref, out_ref):
  x = in_ref[...]
  out_ref[...] = jnp.concatenate([x[:, 0:_TH], x[:, _TH:_TBLK]], axis=0).T


_transpose_call = pl.pallas_call(
    _transpose_body,
    grid=(_TGRID,),
    in_specs=[pl.BlockSpec((_HID, _TBLK), lambda i: (0, i))],
    out_specs=pl.BlockSpec((_TH, _W), lambda i: (i, 0)),
    out_shape=jax.ShapeDtypeStruct((_PROWS, _W), jnp.float32),
)

_REL = 1000
_RBLK = 1024
_RH = _RBLK // 2
_RSHIFT = 9


def _rel_transpose_body(in_ref, out_ref):
  x = in_ref[...]
  out_ref[...] = jnp.concatenate([x[:, 0:_RH], x[:, _RH:_RBLK]], axis=0).T


_rel_transpose_call = pl.pallas_call(
    _rel_transpose_body,
    grid=(1,),
    in_specs=[pl.BlockSpec((_HID, _RBLK), lambda i: (0, i))],
    out_specs=pl.BlockSpec((_RH, _W), lambda i: (i, 0)),
    out_shape=jax.ShapeDtypeStruct((_RH, _W), jnp.float32),
)


def _build_sc_kernel():
  mesh = plsc.VectorSubcoreMesh(core_axis_name="c", subcore_axis_name="s")

  @functools.partial(
      pl.kernel,
      mesh=mesh,
      out_type=jax.ShapeDtypeStruct((_SEQ,), jnp.float32),
      scratch_types=[
          pltpu.VMEM((_ROWS_W,), jnp.int32),
          pltpu.VMEM((_ROWS_W,), jnp.int32),
          pltpu.VMEM((_ROWS_W,), jnp.int32),
          pltpu.VMEM((_ROWS_W,), jnp.int32),
          pltpu.VMEM((_ROWS_W,), jnp.int32),
          pltpu.VMEM((_ROWS_W,), jnp.int32),
          pltpu.VMEM((_CHUNK, _W), jnp.float32),
          pltpu.VMEM((_CHUNK, _W), jnp.float32),
          pltpu.VMEM((_CHUNK, _W), jnp.float32),
          pltpu.VMEM((_CHUNK, _W), jnp.float32),
          pltpu.VMEM((_CHUNK, _W), jnp.float32),
          pltpu.VMEM((_CHUNK, _W), jnp.float32),
          pltpu.VMEM((_CHUNK,), jnp.float32),
          pltpu.SemaphoreType.DMA,
          pltpu.SemaphoreType.DMA,
      ],
  )
  def sc_kernel(h_hbm, t_hbm, r_hbm, ent_hbm, rel_hbm, out_hbm,
                idx_h, idx_t, idx_r, pidx_h, pidx_t, pidx_r,
                rh0, rt0, rr0, rh1, rt1, rr1, ssq_v, sem0, sem1):
    wid = lax.axis_index("s") * _NC + lax.axis_index("c")
    ibase = wid * _ROWS_W
    pltpu.sync_copy(h_hbm.at[pl.ds(ibase, _ROWS_W)], idx_h)
    pltpu.sync_copy(t_hbm.at[pl.ds(ibase, _ROWS_W)], idx_t)
    pltpu.sync_copy(r_hbm.at[pl.ds(ibase, _ROWS_W)], idx_r)
    for s in range(_ROWS_W // _L):
      sl = pl.ds(s * _L, _L)
      eh = idx_h[sl]
      et = idx_t[sl]
      pidx_h[sl] = (lax.shift_left(lax.shift_right_logical(eh, _SHIFT_HI), _SHIFT_LO)
                    | (eh & (_TH - 1)))
      pidx_t[sl] = (lax.shift_left(lax.shift_right_logical(et, _SHIFT_HI), _SHIFT_LO)
                    | (et & (_TH - 1)))
      pidx_r[sl] = idx_r[sl] & (_RH - 1)

    lane = lax.iota(jnp.int32, _L)
    perms = [jnp.bitwise_xor(lane, d) for d in (8, 4, 2, 1)]

    bufs = ((rh0, rt0, rr0, sem0), (rh1, rt1, rr1, sem1))

    def issue(j, b):
      rh, rt, rr, sem = bufs[b]
      csl = pl.ds(j * _CHUNK, _CHUNK)
      return (pltpu.async_copy(ent_hbm.at[pidx_h.at[csl]], rh, sem),
              pltpu.async_copy(ent_hbm.at[pidx_t.at[csl]], rt, sem),
              pltpu.async_copy(rel_hbm.at[pidx_r.at[csl]], rr, sem))

    pending = issue(0, 0)
    for j in range(_NCH):
      rows_h, rows_t, rows_r, _ = bufs[j % 2]
      cur = pending
      if j + 1 < _NCH:
        pending = issue(j + 1, (j + 1) % 2)
      for c in cur:
        c.wait()

      def group_body(g, carry):
        gb = j * _CHUNK + g * _L
        gsl = pl.ds(gb, _L)
        ph = (lax.shift_right_logical(idx_h[gsl], _SHIFT_LO) & 1) * _HID
        pt = (lax.shift_right_logical(idx_t[gsl], _SHIFT_LO) & 1) * _HID
        pr = (lax.shift_right_logical(idx_r[gsl], _RSHIFT) & 1) * _HID
        sv = jnp.zeros((_L,), jnp.float32)
        for k in range(_L):
          i = g * _L + k
          bh, bt, br = ph[k], pt[k], pr[k]
          acc = None
          for q in range(_NSEG):
            o = q * _L
            d = (rows_h[i, pl.ds(bh + o, _L)] + rows_r[i, pl.ds(br + o, _L)]
                 - rows_t[i, pl.ds(bt + o, _L)])
            acc = d * d if acc is None else acc + d * d
          for p in perms:
            acc = acc + _permute(acc, p)
          sv = jnp.where(lane == k, acc, sv)
        ssq_v[pl.ds(g * _L, _L)] = sv
        return carry

      lax.fori_loop(0, _CHUNK // _L, group_body, 0)

      pltpu.sync_copy(ssq_v, out_hbm.at[pl.ds(wid * _ROWS_W + j * _CHUNK, _CHUNK)])

  return sc_kernel


_sc_gather_ssq = _build_sc_kernel()


def _loss_body(ssq_ref, out_ref):
  score = jnp.sqrt(ssq_ref[...])
  p = score[0:1, :]
  n = jnp.mean(score[1:, :], axis=0, keepdims=True)
  out_ref[0, 0] = jnp.sum(jnp.maximum(0.0, p - n + _MARGIN))


_loss_call = pl.pallas_call(
    _loss_body,
    out_shape=jax.ShapeDtypeStruct((1, 1), jnp.float32),
    out_specs=pl.BlockSpec(memory_space=pltpu.SMEM),
)


def kernel(batch_h, batch_t, batch_r, ent_embeddings, rel_embeddings):
  ent2 = _transpose_call(ent_embeddings.T)
  rel2 = _rel_transpose_call(rel_embeddings.T)
  ssq = _sc_gather_ssq(batch_h, batch_t, batch_r, ent2, rel2)
  loss = _loss_call(ssq.reshape(_SEQ // _BATCH, _BATCH))
  return loss[0, 0]


# speedup vs baseline: 1.0034x; 1.0034x over previous
import functools

import jax
import jax.numpy as jnp
from jax import lax
from jax.experimental import pallas as pl
from jax.experimental.pallas import tpu as pltpu
from jax.experimental.pallas import tpu_sc as plsc

_HID = 64
_BATCH = 4096
_SEQ = 24576
_MARGIN = 1.0

_NC = 2
_NS = 16
_L = 16
_NW = _NC * _NS
_ROWS_W = _SEQ // _NW
_CHUNK = 128
_NCH = _ROWS_W // _CHUNK
_NSEG = _HID // _L
_W = 2 * _HID

_ENT = 1000000
_TBLK = 32768
_TH = _TBLK // 2
_SHIFT_HI = 15
_SHIFT_LO = 14
_TGRID = pl.cdiv(_ENT, _TBLK)
_PROWS = _TGRID * _TH


def _permute(v, idx):
  dnums = lax.GatherDimensionNumbers(
      offset_dims=(), collapsed_slice_dims=(0,), start_index_map=(0,))
  return lax.gather(v, idx[:, None], dnums, (1,),
                    mode=lax.GatherScatterMode.PROMISE_IN_BOUNDS)


def _transpose_body(in_ref, out_ref):
  x = in_ref[...]
  out_ref[...] = jnp.concatenate([x[:, 0:_TH], x[:, _TH:_TBLK]], axis=0).T


_transpose_call = pl.pallas_call(
    _transpose_body,
    grid=(_TGRID,),
    in_specs=[pl.BlockSpec((_HID, _TBLK), lambda i: (0, i))],
    out_specs=pl.BlockSpec((_TH, _W), lambda i: (i, 0)),
    out_shape=jax.ShapeDtypeStruct((_PROWS, _W), jnp.float32),
)

def _build_sc_kernel():
  mesh = plsc.VectorSubcoreMesh(core_axis_name="c", subcore_axis_name="s")

  @functools.partial(
      pl.kernel,
      mesh=mesh,
      out_type=jax.ShapeDtypeStruct((_SEQ,), jnp.float32),
      scratch_types=[
          pltpu.VMEM((_ROWS_W,), jnp.int32),
          pltpu.VMEM((_ROWS_W,), jnp.int32),
          pltpu.VMEM((_ROWS_W,), jnp.int32),
          pltpu.VMEM((_ROWS_W,), jnp.int32),
          pltpu.VMEM((_ROWS_W,), jnp.int32),
          pltpu.VMEM((_CHUNK, _W), jnp.float32),
          pltpu.VMEM((_CHUNK, _W), jnp.float32),
          pltpu.VMEM((_CHUNK, _W), jnp.float32),
          pltpu.VMEM((_CHUNK, _W), jnp.float32),
          pltpu.VMEM((_CHUNK, _W), jnp.float32),
          pltpu.VMEM((_CHUNK, _W), jnp.float32),
          pltpu.VMEM((_CHUNK,), jnp.float32),
          pltpu.SemaphoreType.DMA,
          pltpu.SemaphoreType.DMA,
      ],
  )
  def sc_kernel(h_hbm, t_hbm, r_hbm, ent_hbm, rel_hbm, out_hbm,
                idx_h, idx_t, idx_r, pidx_h, pidx_t,
                rh0, rt0, rr0, rh1, rt1, rr1, ssq_v, sem0, sem1):
    wid = lax.axis_index("s") * _NC + lax.axis_index("c")
    ibase = wid * _ROWS_W
    pltpu.sync_copy(h_hbm.at[pl.ds(ibase, _ROWS_W)], idx_h)
    pltpu.sync_copy(t_hbm.at[pl.ds(ibase, _ROWS_W)], idx_t)
    pltpu.sync_copy(r_hbm.at[pl.ds(ibase, _ROWS_W)], idx_r)
    for s in range(_ROWS_W // _L):
      sl = pl.ds(s * _L, _L)
      eh = idx_h[sl]
      et = idx_t[sl]
      pidx_h[sl] = (lax.shift_left(lax.shift_right_logical(eh, _SHIFT_HI), _SHIFT_LO)
                    | (eh & (_TH - 1)))
      pidx_t[sl] = (lax.shift_left(lax.shift_right_logical(et, _SHIFT_HI), _SHIFT_LO)
                    | (et & (_TH - 1)))

    lane = lax.iota(jnp.int32, _L)
    perms = [jnp.bitwise_xor(lane, d) for d in (8, 4, 2, 1)]

    bufs = ((rh0, rt0, rr0, sem0), (rh1, rt1, rr1, sem1))

    def issue(j, b):
      rh, rt, rr, sem = bufs[b]
      csl = pl.ds(j * _CHUNK, _CHUNK)
      return (pltpu.async_copy(ent_hbm.at[pidx_h.at[csl]], rh, sem),
              pltpu.async_copy(ent_hbm.at[pidx_t.at[csl]], rt, sem),
              pltpu.async_copy(rel_hbm.at[idx_r.at[csl]], rr, sem))

    pending = issue(0, 0)
    for j in range(_NCH):
      rows_h, rows_t, rows_r, _ = bufs[j % 2]
      cur = pending
      if j + 1 < _NCH:
        pending = issue(j + 1, (j + 1) % 2)
      for c in cur:
        c.wait()

      def group_body(g, carry):
        gb = j * _CHUNK + g * _L
        gsl = pl.ds(gb, _L)
        ph = (lax.shift_right_logical(idx_h[gsl], _SHIFT_LO) & 1) * _HID
        pt = (lax.shift_right_logical(idx_t[gsl], _SHIFT_LO) & 1) * _HID
        sv = jnp.zeros((_L,), jnp.float32)
        for k in range(_L):
          i = g * _L + k
          bh, bt = ph[k], pt[k]
          acc = None
          for q in range(_NSEG):
            o = q * _L
            d = (rows_h[i, pl.ds(bh + o, _L)] + rows_r[i, pl.ds(o, _L)]
                 - rows_t[i, pl.ds(bt + o, _L)])
            acc = d * d if acc is None else acc + d * d
          for p in perms:
            acc = acc + _permute(acc, p)
          sv = jnp.where(lane == k, acc, sv)
        ssq_v[pl.ds(g * _L, _L)] = sv
        return carry

      lax.fori_loop(0, _CHUNK // _L, group_body, 0)

      pltpu.sync_copy(ssq_v, out_hbm.at[pl.ds(wid * _ROWS_W + j * _CHUNK, _CHUNK)])

  return sc_kernel


_sc_gather_ssq = _build_sc_kernel()


def _loss_body(ssq_ref, out_ref):
  score = jnp.sqrt(ssq_ref[...])
  p = score[0:1, :]
  n = jnp.mean(score[1:, :], axis=0, keepdims=True)
  out_ref[0, 0] = jnp.sum(jnp.maximum(0.0, p - n + _MARGIN))


_loss_call = pl.pallas_call(
    _loss_body,
    out_shape=jax.ShapeDtypeStruct((1, 1), jnp.float32),
    out_specs=pl.BlockSpec(memory_space=pltpu.SMEM),
)


def kernel(batch_h, batch_t, batch_r, ent_embeddings, rel_embeddings):
  ent2 = _transpose_call(ent_embeddings.T)
  rel2 = jnp.pad(rel_embeddings, ((0, 0), (0, _W - _HID)))
  ssq = _sc_gather_ssq(batch_h, batch_t, batch_r, ent2, rel2)
  loss = _loss_call(ssq.reshape(_SEQ // _BATCH, _BATCH))
  return loss[0, 0]
